# baseline (device time: 59912 ns/iter reference)
import jax
import jax.numpy as jnp
from jax import lax
from jax.experimental import pallas as pl
from jax.experimental.pallas import tpu as pltpu

N_DEV = 4
N_TOK = 2048
D_MODEL = 512
D_HID = 1024
N_EXP = 32
E_LOCAL = N_EXP // N_DEV
CAP = 51
B = N_TOK // N_DEV


def kernel(x, router_W, route_idx, expert_W):
    del router_W
    my = lax.axis_index("i")

    e = route_idx[:, 0]
    onehot = e[:, None] == jnp.arange(N_EXP)[None, :]
    pos = jnp.cumsum(onehot.astype(jnp.int32), axis=0)
    rank = jnp.sum(jnp.where(onehot, pos, 0), axis=1)
    keep = rank <= CAP
    gates = (onehot & keep[:, None]).astype(jnp.bfloat16)
    local_gates = lax.dynamic_slice(gates, (0, my * E_LOCAL), (N_TOK, E_LOCAL))

    x_bf = x.astype(jnp.bfloat16)
    w_bf = expert_W.astype(jnp.bfloat16)

    def body(x_ref, g_ref, w_ref, out_ref, part_ref, recv_ref, send_sems, recv_sems):
        my_pos = lax.axis_index("i")

        bar = pltpu.get_barrier_semaphore()
        for q in (1, 2, 3):
            pl.semaphore_signal(
                bar,
                inc=1,
                device_id=((my_pos + q) % N_DEV,),
                device_id_type=pl.DeviceIdType.MESH,
            )
        pl.semaphore_wait(bar, 3)

        def compute_block(dest):
            xb = x_ref[pl.ds(dest * B, B), :]
            gb = g_ref[pl.ds(dest * B, B), :]
            acc = jnp.zeros((B, D_HID), jnp.float32)
            for j in range(E_LOCAL):
                xm = xb * gb[:, j][:, None]
                acc = acc + jnp.dot(
                    xm, w_ref[j], preferred_element_type=jnp.float32
                )
            return acc

        sends = []
        for q in (1, 2, 3):
            dest = (my_pos + q) % N_DEV
            slot = N_DEV - q
            part_ref[slot] = compute_block(dest).astype(jnp.bfloat16)
            rdma = pltpu.make_async_remote_copy(
                src_ref=part_ref.at[slot],
                dst_ref=recv_ref.at[slot],
                send_sem=send_sems.at[slot],
                recv_sem=recv_sems.at[slot],
                device_id=(dest,),
                device_id_type=pl.DeviceIdType.MESH,
            )
            rdma.start()
            sends.append(rdma)

        out_ref[:, :] = compute_block(my_pos)

        for d in (1, 2, 3):
            src_dev = (my_pos + d) % N_DEV
            recv = pltpu.make_async_remote_copy(
                src_ref=part_ref.at[d],
                dst_ref=recv_ref.at[d],
                send_sem=send_sems.at[d],
                recv_sem=recv_sems.at[d],
                device_id=(src_dev,),
                device_id_type=pl.DeviceIdType.MESH,
            )
            recv.wait_recv()
            out_ref[:, :] += recv_ref[d].astype(jnp.float32)

        for rdma in sends:
            rdma.wait_send()

    return pl.pallas_call(
        body,
        out_shape=jax.ShapeDtypeStruct((B, D_HID), jnp.float32),
        in_specs=[
            pl.BlockSpec(memory_space=pltpu.VMEM),
            pl.BlockSpec(memory_space=pltpu.VMEM),
            pl.BlockSpec(memory_space=pltpu.VMEM),
        ],
        out_specs=pl.BlockSpec(memory_space=pltpu.VMEM),
        scratch_shapes=[
            pltpu.VMEM((N_DEV, B, D_HID), jnp.bfloat16),
            pltpu.VMEM((N_DEV, B, D_HID), jnp.bfloat16),
            pltpu.SemaphoreType.DMA((N_DEV,)),
            pltpu.SemaphoreType.DMA((N_DEV,)),
        ],
        compiler_params=pltpu.CompilerParams(collective_id=0),
    )(x_bf, local_gates, w_bf)


# device time: 35560 ns/iter; 1.6848x vs baseline; 1.6848x over previous
import os

import jax
import jax.numpy as jnp
from jax import lax
from jax.experimental import pallas as pl
from jax.experimental.pallas import tpu as pltpu

ABLATE = os.environ.get("ABLATE", "")
CHUNK_E = int(os.environ.get("CHUNK_E", "1"))

N_DEV = 4
N_TOK = 2048
D_MODEL = 512
D_HID = 1024
N_EXP = 32
E_LOCAL = N_EXP // N_DEV
CAP = 51
CAP_PAD = 64
S_LOCAL = E_LOCAL * CAP_PAD
B = N_TOK // N_DEV
PAIR = CHUNK_E * CAP_PAD
N_PAIR = E_LOCAL // CHUNK_E
DROP = jnp.int32(9999)


def kernel(x, router_W, route_idx, expert_W):
    del router_W
    my = lax.axis_index("i")

    e = route_idx[:, 0]
    e3 = e.reshape(N_TOK // 32, 32, 1)
    oh3 = e3 == jnp.arange(N_EXP)[None, None, :]
    within = jnp.cumsum(oh3.astype(jnp.int32), axis=1)
    gsum = within[:, -1, :]
    prefix = jnp.cumsum(gsum, axis=0) - gsum
    pos3 = within + prefix[:, None, :]
    rank = jnp.sum(jnp.where(oh3, pos3, 0), axis=-1).reshape(N_TOK)
    slot = jnp.where(rank <= CAP, e * CAP_PAD + rank - 1, DROP).astype(jnp.int32)
    slot_row = slot.reshape(1, N_TOK)
    slot_col = lax.dynamic_slice(slot, (my * B,), (B,)).reshape(B, 1)

    def body(
        x32_ref, srow_ref, scol_ref, w32_ref, out_ref,
        x32v_ref, x_ref, wbuf_ref, yq_ref, sc_ref, rq_ref, rsc_ref,
        xdma_sem, wdma_sems, send_sems, recv_sems, ssend_sems, srecv_sems,
    ):
        my_pos = lax.axis_index("i")
        comm = ABLATE != "nocomm"
        mmul = ABLATE != "nommul"

        xdma = pltpu.make_async_copy(x32_ref, x32v_ref, xdma_sem)
        xdma.start()
        wdmas = [None] * E_LOCAL

        def start_wdma(j):
            wdmas[j] = pltpu.make_async_copy(
                w32_ref.at[j], wbuf_ref.at[j % 2], wdma_sems.at[j % 2]
            )
            wdmas[j].start()

        start_wdma(0)

        if comm:
            bar = pltpu.get_barrier_semaphore()
            for q in (1, 2, 3):
                pl.semaphore_signal(
                    bar,
                    inc=1,
                    device_id=((my_pos + q) % N_DEV,),
                    device_id_type=pl.DeviceIdType.MESH,
                )
            pl.semaphore_wait(bar, 3)

        sends = []
        scales = []

        if mmul:
            xdma.wait()
            x_ref[:, :] = x32v_ref[:, :].astype(jnp.bfloat16)
            sl = jax.lax.broadcasted_iota(jnp.int32, (S_LOCAL, N_TOK), 0)
            disp = (srow_ref[:, :] - my_pos * S_LOCAL == sl).astype(jnp.bfloat16)
            xc = jnp.dot(
                disp, x_ref[:, :], preferred_element_type=jnp.float32
            ).astype(jnp.bfloat16)

        for j in range(E_LOCAL):
            if mmul:
                wdmas[j].wait()
                if j + 1 < E_LOCAL:
                    start_wdma(j + 1)
                yj = jnp.dot(
                    xc[j * CAP_PAD : (j + 1) * CAP_PAD, :],
                    wbuf_ref[j % 2].astype(jnp.bfloat16),
                    preferred_element_type=jnp.float32,
                )
                m = jnp.maximum(jnp.max(jnp.abs(yj)), 1e-20)
                yq_ref[pl.ds(j * CAP_PAD, CAP_PAD), :] = jnp.round(
                    yj * (127.0 / m)
                ).astype(jnp.int8)
                scales.append(m / 127.0)
            else:
                yq_ref[pl.ds(j * CAP_PAD, CAP_PAD), :] = jnp.zeros(
                    (CAP_PAD, D_HID), jnp.int8
                )
                scales.append(jnp.float32(1.0))
            if comm and j % CHUNK_E == CHUNK_E - 1:
                p = j // CHUNK_E
                for q in (1, 2, 3):
                    d = N_DEV - q
                    rdma = pltpu.make_async_remote_copy(
                        src_ref=yq_ref.at[pl.ds(p * PAIR, PAIR)],
                        dst_ref=rq_ref.at[pl.ds(d * S_LOCAL + p * PAIR, PAIR)],
                        send_sem=send_sems.at[d * N_PAIR + p],
                        recv_sem=recv_sems.at[d * N_PAIR + p],
                        device_id=((my_pos + q) % N_DEV,),
                        device_id_type=pl.DeviceIdType.MESH,
                    )
                    rdma.start()
                    sends.append(rdma)

        if not mmul:
            xdma.wait()
            wdmas[0].wait()

        lane = jax.lax.broadcasted_iota(jnp.int32, (1, 128), 1)
        sc = jnp.zeros((1, 128), jnp.float32)
        for j, s in enumerate(scales):
            sc = jnp.where(lane == j, s, sc)
        sc_ref[:, :] = sc
        for q in (1, 2, 3) if comm else ():
            d = N_DEV - q
            rdma = pltpu.make_async_remote_copy(
                src_ref=sc_ref,
                dst_ref=rsc_ref.at[d],
                send_sem=ssend_sems.at[d],
                recv_sem=srecv_sems.at[d],
                device_id=((my_pos + q) % N_DEV,),
                device_id_type=pl.DeviceIdType.MESH,
            )
            rdma.start()
            sends.append(rdma)

        ct = jax.lax.broadcasted_iota(jnp.int32, (B, S_LOCAL), 1)

        def c_mat(src_dev):
            match = scol_ref[:, :] - src_dev * S_LOCAL == ct
            return match.astype(jnp.int32).astype(jnp.int8)

        def col_scale(src_dev, scale_at):
            rel = scol_ref[:, :] - src_dev * S_LOCAL
            cs = jnp.zeros((B, 1), jnp.float32)
            for j in range(E_LOCAL):
                inblk = (rel >= j * CAP_PAD) & (rel < (j + 1) * CAP_PAD)
                cs = jnp.where(inblk, scale_at(j), cs)
            return cs

        def accum_int8(src_dev, q2d, scale_at, first):
            r = jnp.dot(c_mat(src_dev), q2d, preferred_element_type=jnp.int32)
            contrib = r.astype(jnp.float32) * col_scale(src_dev, scale_at)
            if first:
                out_ref[:, :] = contrib
            else:
                out_ref[:, :] += contrib

        if mmul:
            accum_int8(my_pos, yq_ref[:, :], lambda j: scales[j], True)
        else:
            out_ref[:, :] = jnp.zeros((B, D_HID), jnp.float32)

        for d in (1, 2, 3) if comm else ():
            src_dev = (my_pos + d) % N_DEV
            for p in range(N_PAIR):
                recv = pltpu.make_async_remote_copy(
                    src_ref=yq_ref.at[pl.ds(p * PAIR, PAIR)],
                    dst_ref=rq_ref.at[pl.ds(d * S_LOCAL + p * PAIR, PAIR)],
                    send_sem=send_sems.at[d * N_PAIR + p],
                    recv_sem=recv_sems.at[d * N_PAIR + p],
                    device_id=(src_dev,),
                    device_id_type=pl.DeviceIdType.MESH,
                )
                recv.wait_recv()
            srecv = pltpu.make_async_remote_copy(
                src_ref=sc_ref,
                dst_ref=rsc_ref.at[d],
                send_sem=ssend_sems.at[d],
                recv_sem=srecv_sems.at[d],
                device_id=(src_dev,),
                device_id_type=pl.DeviceIdType.MESH,
            )
            srecv.wait_recv()
            if not mmul:
                continue
            accum_int8(
                src_dev,
                rq_ref[pl.ds(d * S_LOCAL, S_LOCAL), :],
                lambda j, _d=d: rsc_ref[_d][0:1, j : j + 1],
                False,
            )

        for rdma in sends:
            rdma.wait_send()

    return pl.pallas_call(
        body,
        out_shape=jax.ShapeDtypeStruct((B, D_HID), jnp.float32),
        in_specs=[
            pl.BlockSpec(memory_space=pl.ANY),
            pl.BlockSpec(memory_space=pltpu.VMEM),
            pl.BlockSpec(memory_space=pltpu.VMEM),
            pl.BlockSpec(memory_space=pl.ANY),
        ],
        out_specs=pl.BlockSpec(memory_space=pltpu.VMEM),
        scratch_shapes=[
            pltpu.VMEM((N_TOK, D_MODEL), jnp.float32),
            pltpu.VMEM((N_TOK, D_MODEL), jnp.bfloat16),
            pltpu.VMEM((2, D_MODEL, D_HID), jnp.float32),
            pltpu.VMEM((S_LOCAL, D_HID), jnp.int8),
            pltpu.VMEM((1, 128), jnp.float32),
            pltpu.VMEM((N_DEV * S_LOCAL, D_HID), jnp.int8),
            pltpu.VMEM((N_DEV, 1, 128), jnp.float32),
            pltpu.SemaphoreType.DMA,
            pltpu.SemaphoreType.DMA((2,)),
            pltpu.SemaphoreType.DMA((N_DEV * N_PAIR,)),
            pltpu.SemaphoreType.DMA((N_DEV * N_PAIR,)),
            pltpu.SemaphoreType.DMA((N_DEV,)),
            pltpu.SemaphoreType.DMA((N_DEV,)),
        ],
        compiler_params=pltpu.CompilerParams(
            collective_id=None if ABLATE == "nocomm" else 0,
            vmem_limit_bytes=64 * 1024 * 1024,
        ),
    )(x, slot_row, slot_col, expert_W)
